# baseline (speedup 1.0000x reference)
import functools

import jax
import jax.numpy as jnp
from jax import lax
from jax.experimental import pallas as pl
from jax.experimental.pallas import tpu as pltpu
from jax.experimental.pallas import tpu_sc as plsc

N_FIELDS = 26
VOCAB = 100000
EMBED_DIM = 32
DENSE_DIM = 13
BATCH = 4096
NV = N_FIELDS + 1
ROWS_PER_128 = 128 // EMBED_DIM

_B_PER_W = 128
_SUB_B = 16
_SUB_N = _SUB_B * N_FIELDS
_N_SUB = _B_PER_W // _SUB_B
_L = 16


def _make_sc_gather():
    info = plsc.get_sparse_core_info()
    NC, NS = info.num_cores, info.num_subcores
    NW = NC * NS
    assert BATCH == NW * _B_PER_W
    n_per_w = _B_PER_W * N_FIELDS
    mesh = plsc.VectorSubcoreMesh(core_axis_name="c", subcore_axis_name="s")

    @functools.partial(
        pl.kernel,
        mesh=mesh,
        out_type=jax.ShapeDtypeStruct((BATCH, N_FIELDS * EMBED_DIM), jnp.float32),
        scratch_types=[
            pltpu.VMEM((n_per_w,), jnp.int32),
            pltpu.VMEM((n_per_w,), jnp.int32),
            pltpu.VMEM((_SUB_N,), jnp.int32),
            pltpu.VMEM((_SUB_N,), jnp.int32),
            pltpu.VMEM((_SUB_N, 128), jnp.float32),
            pltpu.VMEM((_SUB_B, N_FIELDS * EMBED_DIM), jnp.float32),
            pltpu.SemaphoreType.DMA,
        ],
        compiler_params=pltpu.CompilerParams(needs_layout_passes=False),
    )
    def gather_k(tab_hbm, idx_hbm, sub_hbm, orow_hbm, ocol_hbm, out_hbm,
                 idx_v, sub_v, orow_v, ocol_v, buf_v, strip_v, sem):
        wid = lax.axis_index("s") * NC + lax.axis_index("c")
        base_n = wid * n_per_w
        pltpu.sync_copy(idx_hbm.at[pl.ds(base_n, n_per_w)], idx_v)
        pltpu.sync_copy(sub_hbm.at[pl.ds(base_n, n_per_w)], sub_v)
        pltpu.sync_copy(orow_hbm, orow_v)
        pltpu.sync_copy(ocol_hbm, ocol_v)
        lane = lax.iota(jnp.int32, _L)

        for c in range(_N_SUB):
            pltpu.async_copy(
                tab_hbm.at[idx_v.at[pl.ds(c * _SUB_N, _SUB_N)]], buf_v, sem
            ).wait()

            def group_body(g, _):
                nn = g * _L
                sub16 = sub_v[pl.ds(c * _SUB_N + nn, _L)]
                row16 = orow_v[pl.ds(nn, _L)]
                col16 = ocol_v[pl.ds(nn, _L)]
                src_row = nn + lane
                for d in range(EMBED_DIM):
                    vals = plsc.load_gather(buf_v, [src_row, sub16 + d])
                    plsc.store_scatter(strip_v, [row16, col16 + d], vals)
                return _

            lax.fori_loop(0, _SUB_N // _L, group_body, 0)
            pltpu.sync_copy(
                strip_v, out_hbm.at[pl.ds(wid * _B_PER_W + c * _SUB_B, _SUB_B)]
            )

    return gather_k


_sc_gather = _make_sc_gather()


def _dense_body(x_ref, emb_ref, bW0, bb0, bW1, bb1, bW2, bb2,
                tW0t, tb0c, tW1t, tb1c, tW2t, tb2c, out_ref):
    x = x_ref[...]
    h = jnp.maximum(jnp.dot(x, bW0[...], preferred_element_type=jnp.float32) + bb0[...], 0.0)
    h = jnp.maximum(jnp.dot(h, bW1[...], preferred_element_type=jnp.float32) + bb1[...], 0.0)
    h = jnp.maximum(jnp.dot(h, bW2[...], preferred_element_type=jnp.float32) + bb2[...], 0.0)
    feats = jnp.concatenate([h, emb_ref[...]], axis=1)
    ft = feats.T
    f3 = ft.reshape(NV, EMBED_DIM, ft.shape[1])
    parts = []
    for i in range(1, NV):
        parts.append(jnp.sum(f3[:i] * f3[i][None], axis=1))
    inter_t = jnp.concatenate(parts, axis=0)
    top_t = jnp.concatenate([ft[:EMBED_DIM], inter_t], axis=0)
    t = jnp.maximum(jnp.dot(tW0t[...], top_t, preferred_element_type=jnp.float32) + tb0c[...], 0.0)
    t = jnp.maximum(jnp.dot(tW1t[...], t, preferred_element_type=jnp.float32) + tb1c[...], 0.0)
    o = jnp.dot(tW2t[...], t, preferred_element_type=jnp.float32) + tb2c[...]
    out_ref[...] = 1.0 / (1.0 + jnp.exp(-o))


def _dense_call(x, emb2, bW0, bb0, bW1, bb1, bW2, bb2,
                tW0t, tb0c, tW1t, tb1c, tW2t, tb2c):
    Bblk = 512
    grid = (BATCH // Bblk,)
    full = lambda a: pl.BlockSpec(a.shape, lambda i: (0,) * a.ndim)
    ws = [bW0, bb0, bW1, bb1, bW2, bb2, tW0t, tb0c, tW1t, tb1c, tW2t, tb2c]
    out = pl.pallas_call(
        _dense_body,
        grid=grid,
        in_specs=[
            pl.BlockSpec((Bblk, DENSE_DIM), lambda i: (i, 0)),
            pl.BlockSpec((Bblk, N_FIELDS * EMBED_DIM), lambda i: (i, 0)),
        ] + [full(w) for w in ws],
        out_specs=pl.BlockSpec((1, Bblk), lambda i: (0, i)),
        out_shape=jax.ShapeDtypeStruct((1, BATCH), jnp.float32),
    )(x, emb2, *ws)
    return out.reshape(BATCH, 1)


def kernel(dense_x, sparse_indices, tables, bW0, bb0, bW1, bb1, bW2, bb2,
           tW0, tb0, tW1, tb1, tW2, tb2):
    idx = sparse_indices.astype(jnp.int32)
    flat_idx = (idx + jnp.arange(N_FIELDS, dtype=jnp.int32)[None, :] * VOCAB).reshape(-1)
    row_idx = flat_idx // ROWS_PER_128
    sub_off = (flat_idx % ROWS_PER_128) * EMBED_DIM
    nn = jnp.arange(_SUB_N, dtype=jnp.int32)
    orow = nn // N_FIELDS
    ocol = (nn % N_FIELDS) * EMBED_DIM
    tab128 = tables.reshape(N_FIELDS * VOCAB // ROWS_PER_128, 128)
    emb2 = _sc_gather(tab128, row_idx, sub_off, orow, ocol)
    return _dense_call(
        dense_x, emb2,
        bW0, bb0.reshape(1, -1), bW1, bb1.reshape(1, -1), bW2, bb2.reshape(1, -1),
        tW0.T, tb0.reshape(-1, 1), tW1.T, tb1.reshape(-1, 1), tW2.T, tb2.reshape(-1, 1),
    )


# speedup vs baseline: 1.9894x; 1.9894x over previous
import functools

import jax
import jax.numpy as jnp
from jax import lax
from jax.experimental import pallas as pl
from jax.experimental.pallas import tpu as pltpu
from jax.experimental.pallas import tpu_sc as plsc

N_FIELDS = 26
VOCAB = 100000
EMBED_DIM = 32
DENSE_DIM = 13
BATCH = 4096
NV = N_FIELDS + 1
_B_PER_W = 128
_CHUNK = EMBED_DIM * _B_PER_W
_L = 16


def _make_sc_gather():
    info = plsc.get_sparse_core_info()
    NC, NS = info.num_cores, info.num_subcores
    NW = NC * NS
    assert BATCH == NW * _B_PER_W
    mesh = plsc.VectorSubcoreMesh(core_axis_name="c", subcore_axis_name="s")
    FD = N_FIELDS * EMBED_DIM

    @functools.partial(
        pl.kernel,
        mesh=mesh,
        out_type=jax.ShapeDtypeStruct((NW, FD * _B_PER_W), jnp.float32),
        scratch_types=[
            pltpu.VMEM((_B_PER_W,), jnp.int32),
            pltpu.VMEM((2, _CHUNK), jnp.int32),
            pltpu.VMEM((FD * _B_PER_W,), jnp.float32),
            pltpu.SemaphoreType.DMA,
        ],
        compiler_params=pltpu.CompilerParams(use_tc_tiling_on_sc=False),
    )
    def gather_k(tab_hbm, vt_hbm, out_hbm, vv, idx_v, strip_v, sem):
        wid = lax.axis_index("s") * NC + lax.axis_index("c")
        b0 = wid * _B_PER_W
        for f in range(N_FIELDS):
            pltpu.sync_copy(vt_hbm.at[f, pl.ds(b0, _B_PER_W)], vv)
            buf = f % 2
            for l8 in range(_B_PER_W // _L):
                v16 = vv[pl.ds(l8 * _L, _L)]
                for d in range(EMBED_DIM):
                    idx_v[buf, pl.ds(d * _B_PER_W + l8 * _L, _L)] = (
                        v16 + (f * EMBED_DIM + d) * VOCAB
                    )
            pltpu.async_copy(
                tab_hbm.at[idx_v.at[buf]],
                strip_v.at[pl.ds(f * _CHUNK, _CHUNK)],
                sem,
            )
        pltpu.make_async_copy(
            tab_hbm.at[pl.ds(0, FD * _B_PER_W)], strip_v, sem
        ).wait()
        pltpu.sync_copy(strip_v, out_hbm.at[wid])

    return gather_k


_sc_gather = _make_sc_gather()


def _dense_body(xt_ref, embt_ref, bW0t, bb0c, bW1t, bb1c, bW2t, bb2c,
                tW0t, tb0c, tW1t, tb1c, tW2t, tb2c, out_ref):
    x = xt_ref[...]
    h = jnp.maximum(jnp.dot(bW0t[...], x, preferred_element_type=jnp.float32) + bb0c[...], 0.0)
    h = jnp.maximum(jnp.dot(bW1t[...], h, preferred_element_type=jnp.float32) + bb1c[...], 0.0)
    h = jnp.maximum(jnp.dot(bW2t[...], h, preferred_element_type=jnp.float32) + bb2c[...], 0.0)
    ft = jnp.concatenate([h, embt_ref[...]], axis=0)
    f3 = ft.reshape(NV, EMBED_DIM, ft.shape[1])
    parts = []
    for i in range(1, NV):
        parts.append(jnp.sum(f3[:i] * f3[i][None], axis=1))
    inter_t = jnp.concatenate(parts, axis=0)
    top_t = jnp.concatenate([h, inter_t], axis=0)
    t = jnp.maximum(jnp.dot(tW0t[...], top_t, preferred_element_type=jnp.float32) + tb0c[...], 0.0)
    t = jnp.maximum(jnp.dot(tW1t[...], t, preferred_element_type=jnp.float32) + tb1c[...], 0.0)
    o = jnp.dot(tW2t[...], t, preferred_element_type=jnp.float32) + tb2c[...]
    out_ref[...] = 1.0 / (1.0 + jnp.exp(-o))


def _dense_call(xt, embt, *ws):
    Bblk = 512
    grid = (BATCH // Bblk,)
    full = lambda a: pl.BlockSpec(a.shape, lambda i: (0,) * a.ndim)
    out = pl.pallas_call(
        _dense_body,
        grid=grid,
        in_specs=[
            pl.BlockSpec((DENSE_DIM, Bblk), lambda i: (0, i)),
            pl.BlockSpec((N_FIELDS * EMBED_DIM, Bblk), lambda i: (0, i)),
        ] + [full(w) for w in ws],
        out_specs=pl.BlockSpec((1, Bblk), lambda i: (0, i)),
        out_shape=jax.ShapeDtypeStruct((1, BATCH), jnp.float32),
    )(xt, embt, *ws)
    return out.reshape(BATCH, 1)


def kernel(dense_x, sparse_indices, tables, bW0, bb0, bW1, bb1, bW2, bb2,
           tW0, tb0, tW1, tb1, tW2, tb2):
    tab1d = jnp.transpose(tables, (0, 2, 1)).reshape(-1)
    vt = jnp.transpose(sparse_indices.astype(jnp.int32))
    raw = _sc_gather(tab1d, vt)
    embt = (
        raw.reshape(32, N_FIELDS * EMBED_DIM, _B_PER_W)
        .transpose(1, 0, 2)
        .reshape(N_FIELDS * EMBED_DIM, BATCH)
    )
    return _dense_call(
        dense_x.T, embt,
        bW0.T, bb0.reshape(-1, 1), bW1.T, bb1.reshape(-1, 1), bW2.T, bb2.reshape(-1, 1),
        tW0.T, tb0.reshape(-1, 1), tW1.T, tb1.reshape(-1, 1), tW2.T, tb2.reshape(-1, 1),
    )
